# baseline (speedup 1.0000x reference)
import functools

import jax
import jax.numpy as jnp
from jax import lax
from jax.experimental import pallas as pl
from jax.experimental.pallas import tpu as pltpu
from jax.experimental.pallas import tpu_sc as plsc

N = 10000
E = 320000
D = 128

NC = 2
NS = 16
NW = NC * NS
EPW = E // NW
CE = 125
CH = EPW // CE
NP = 10240
RPT = NP // NS
ZR = 40
NSB = 4
SBC = CH // NSB

_mesh = plsc.VectorSubcoreMesh(
    core_axis_name="c", subcore_axis_name="s", num_cores=NC, num_subcores=NS
)


@functools.partial(
    pl.kernel,
    out_type=jax.ShapeDtypeStruct((NC, NP, D), jnp.float32),
    mesh=_mesh,
    scratch_types=[
        pltpu.VMEM((SBC, CE), jnp.int32),
        pltpu.VMEM((SBC, CE), jnp.int32),
        pltpu.VMEM((CE, D), jnp.float32),
        pltpu.VMEM((CE, D), jnp.float32),
        pltpu.VMEM_SHARED((NP, D), jnp.float32),
        pltpu.SemaphoreType.DMA,
        pltpu.SemaphoreType.DMA,
    ],
)
def _sc_aggregate(x_hbm, src_hbm, dst_hbm, zsum_hbm,
                  psum_hbm,
                  src_v, dst_v, buf0, buf1, acc, semg, sems):
    c = lax.axis_index("c")
    s = lax.axis_index("s")
    w = s * NC + c
    r0 = s * RPT
    bufs = (buf0, buf1)

    def zero_acc():
        pltpu.sync_copy(zsum_hbm, buf0.at[pl.ds(0, ZR)])
        for t in range(RPT // ZR):
            pltpu.sync_copy(buf0.at[pl.ds(0, ZR)], acc.at[pl.ds(r0 + t * ZR, ZR)])

    def drain(out_hbm):
        halves = (buf0.at[pl.ds(0, ZR)], buf0.at[pl.ds(ZR, ZR)])
        wd = None
        for t in range(RPT // ZR):
            rr = r0 + t * ZR
            pltpu.sync_copy(acc.at[pl.ds(rr, ZR)], halves[t % 2])
            if wd is not None:
                wd.wait()
            wd = pltpu.async_copy(halves[t % 2], out_hbm.at[c, pl.ds(rr, ZR)], semg)
        wd.wait()

    zero_acc()
    plsc.subcore_barrier()

    def sb_body(sb, carry):
        pltpu.sync_copy(src_hbm.at[w * NSB + sb], src_v)
        pltpu.sync_copy(dst_hbm.at[w * NSB + sb], dst_v)
        gat = pltpu.async_copy(x_hbm.at[src_v.at[0]], bufs[0], semg)
        sca = [None, None]
        for k in range(SBC):
            gat.wait()
            if k + 1 < SBC:
                if sca[(k + 1) % 2] is not None:
                    sca[(k + 1) % 2].wait()
                    sca[(k + 1) % 2] = None
                gat = pltpu.async_copy(
                    x_hbm.at[src_v.at[k + 1]], bufs[(k + 1) % 2], semg)
            sca[k % 2] = pltpu.async_copy(
                bufs[k % 2], acc.at[dst_v.at[k]], sems, add=True)
        for d in sca:
            if d is not None:
                d.wait()
        return carry

    lax.fori_loop(0, NSB, sb_body, 0)
    plsc.subcore_barrier()
    drain(psum_hbm)


BKE = 4000
HI = NP // D


def _hist_body(d_ref, o_ref):
    i = pl.program_id(0)
    d = d_ref[...]
    hi = d >> 7
    lo = d & 127
    oh_hi = (hi == lax.broadcasted_iota(jnp.int32, (1, HI), 1)).astype(jnp.bfloat16)
    oh_lo = (lo == lax.broadcasted_iota(jnp.int32, (1, D), 1)).astype(jnp.bfloat16)
    p = lax.dot_general(oh_hi, oh_lo, (((0,), (0,)), ((), ())),
                        preferred_element_type=jnp.float32)

    @pl.when(i == 0)
    def _():
        o_ref[...] = p

    @pl.when(i > 0)
    def _():
        o_ref[...] += p


_tc_hist = pl.pallas_call(
    _hist_body,
    grid=(E // BKE,),
    in_specs=[pl.BlockSpec((BKE, 1), lambda i: (i, 0))],
    out_specs=pl.BlockSpec((HI, D), lambda i: (0, 0)),
    out_shape=jax.ShapeDtypeStruct((HI, D), jnp.float32),
)


BN = 1024


def _tc_body(ps, pc, x, wrz, woz, bz, wrh, woh, bh, o):
    ssum = ps[0] + ps[1]
    inv = 1.0 / jnp.maximum(pc[...], 1.0)
    aggr = ssum * inv
    xv = x[...]
    z = jax.nn.sigmoid(
        jnp.dot(aggr, wrz[...], preferred_element_type=jnp.float32)
        + jnp.dot(xv, woz[...], preferred_element_type=jnp.float32)
        + bz[...]
    )
    ht = jnp.tanh(
        jnp.dot(aggr, wrh[...], preferred_element_type=jnp.float32)
        + jnp.dot(xv, woh[...], preferred_element_type=jnp.float32)
        + bh[...]
    )
    o[...] = (1.0 - z) * ht


_tc_combine = pl.pallas_call(
    _tc_body,
    grid=(NP // BN,),
    in_specs=[
        pl.BlockSpec((NC, BN, D), lambda i: (0, i, 0)),
        pl.BlockSpec((BN, 1), lambda i: (i, 0)),
        pl.BlockSpec((BN, D), lambda i: (i, 0)),
        pl.BlockSpec((D, D), lambda i: (0, 0)),
        pl.BlockSpec((D, D), lambda i: (0, 0)),
        pl.BlockSpec((1, D), lambda i: (0, 0)),
        pl.BlockSpec((D, D), lambda i: (0, 0)),
        pl.BlockSpec((D, D), lambda i: (0, 0)),
        pl.BlockSpec((1, D), lambda i: (0, 0)),
    ],
    out_specs=pl.BlockSpec((BN, D), lambda i: (i, 0)),
    out_shape=jax.ShapeDtypeStruct((NP, D), jnp.float32),
)


def kernel(X, edge_index, W_rel_z, W_root_z, b_z, W_rel_r, W_root_r, b_r,
           W_rel_h, W_root_h, b_h):
    src = edge_index[0].reshape(NW * NSB, SBC, CE)
    dst = edge_index[1].reshape(NW * NSB, SBC, CE)
    zsum = jnp.zeros((ZR, D), jnp.float32)
    x_pad = jnp.concatenate([X, jnp.zeros((NP - N, D), jnp.float32)])
    psum = _sc_aggregate(x_pad, src, dst, zsum)
    cnt = _tc_hist(edge_index[1].reshape(E, 1)).reshape(NP, 1)
    out = _tc_combine(
        psum, cnt, x_pad,
        W_rel_z[:D], W_root_z[:D], b_z.reshape(1, D),
        W_rel_h[:D], W_root_h[:D], b_h.reshape(1, D),
    )
    return out[:N]


# speedup vs baseline: 1.1097x; 1.1097x over previous
import functools

import jax
import jax.numpy as jnp
from jax import lax
from jax.experimental import pallas as pl
from jax.experimental.pallas import tpu as pltpu
from jax.experimental.pallas import tpu_sc as plsc

N = 10000
E = 320000
D = 128

NC = 2
NS = 16
NW = NC * NS
EPW = E // NW
CE = 125
CH = EPW // CE
NP = 10240
RPT = NP // NS
ZR = 40
NSB = 4
SBC = CH // NSB

_mesh = plsc.VectorSubcoreMesh(
    core_axis_name="c", subcore_axis_name="s", num_cores=NC, num_subcores=NS
)


@functools.partial(
    pl.kernel,
    out_type=(
        jax.ShapeDtypeStruct((NC, NP, D), jnp.float32),
        jax.ShapeDtypeStruct((NC, NP, D), jnp.float32),
    ),
    mesh=_mesh,
    scratch_types=[
        pltpu.VMEM((SBC, CE), jnp.int32),
        pltpu.VMEM((SBC, CE), jnp.int32),
        pltpu.VMEM((CE, D), jnp.float32),
        pltpu.VMEM((CE, D), jnp.float32),
        pltpu.VMEM_SHARED((NP, D), jnp.float32),
        pltpu.SemaphoreType.DMA,
        pltpu.SemaphoreType.DMA,
    ],
)
def _sc_aggregate(x_hbm, src_hbm, dst_hbm, zsum_hbm, ones_hbm,
                  psum_hbm, pcnt_hbm,
                  src_v, dst_v, buf0, buf1, acc, semg, sems):
    c = lax.axis_index("c")
    s = lax.axis_index("s")
    w = s * NC + c
    r0 = s * RPT
    bufs = (buf0, buf1)

    def zero_acc():
        pltpu.sync_copy(zsum_hbm, buf0.at[pl.ds(0, ZR)])
        for t in range(RPT // ZR):
            pltpu.sync_copy(buf0.at[pl.ds(0, ZR)], acc.at[pl.ds(r0 + t * ZR, ZR)])

    def drain(out_hbm, rezero):
        if rezero:
            pltpu.sync_copy(zsum_hbm, buf1.at[pl.ds(0, ZR)])
        halves = (buf0.at[pl.ds(0, ZR)], buf0.at[pl.ds(ZR, ZR)])
        nst = RPT // ZR
        wd = None
        for t in range(nst):
            rr = r0 + t * ZR
            pltpu.sync_copy(acc.at[pl.ds(rr, ZR)], halves[t % 2])
            if wd is not None:
                wd.wait()
            wd = pltpu.async_copy(halves[t % 2], out_hbm.at[c, pl.ds(rr, ZR)], semg)
            if rezero:
                pltpu.sync_copy(buf1.at[pl.ds(0, ZR)], acc.at[pl.ds(rr, ZR)])
        wd.wait()

    zero_acc()
    plsc.subcore_barrier()

    def sb_body(sb, carry):
        pltpu.sync_copy(src_hbm.at[w * NSB + sb], src_v)
        pltpu.sync_copy(dst_hbm.at[w * NSB + sb], dst_v)
        gat = pltpu.async_copy(x_hbm.at[src_v.at[0]], bufs[0], semg)
        sca = [None, None]
        for k in range(SBC):
            gat.wait()
            if k + 1 < SBC:
                if sca[(k + 1) % 2] is not None:
                    sca[(k + 1) % 2].wait()
                    sca[(k + 1) % 2] = None
                gat = pltpu.async_copy(
                    x_hbm.at[src_v.at[k + 1]], bufs[(k + 1) % 2], semg)
            sca[k % 2] = pltpu.async_copy(
                bufs[k % 2], acc.at[dst_v.at[k]], sems, add=True)
        for d in sca:
            if d is not None:
                d.wait()
        return carry

    lax.fori_loop(0, NSB, sb_body, 0)
    plsc.subcore_barrier()
    drain(psum_hbm, rezero=True)
    plsc.subcore_barrier()

    pltpu.sync_copy(ones_hbm, buf1)

    def sb_body2(sb, carry):
        pltpu.sync_copy(dst_hbm.at[w * NSB + sb], dst_v)
        descs = [
            pltpu.async_copy(buf1, acc.at[dst_v.at[k]], sems, add=True)
            for k in range(SBC)
        ]
        for d in descs:
            d.wait()
        return carry

    lax.fori_loop(0, NSB, sb_body2, 0)
    plsc.subcore_barrier()
    drain(pcnt_hbm, rezero=False)


BN = 1024


def _tc_body(ps, pc, x, wrz, woz, bz, wrh, woh, bh, o):
    ssum = ps[0] + ps[1]
    cnt = pc[0] + pc[1]
    inv = 1.0 / jnp.maximum(cnt[:, 0:1], 1.0)
    aggr = ssum * inv
    xv = x[...]
    z = jax.nn.sigmoid(
        jnp.dot(aggr, wrz[...], preferred_element_type=jnp.float32)
        + jnp.dot(xv, woz[...], preferred_element_type=jnp.float32)
        + bz[...]
    )
    ht = jnp.tanh(
        jnp.dot(aggr, wrh[...], preferred_element_type=jnp.float32)
        + jnp.dot(xv, woh[...], preferred_element_type=jnp.float32)
        + bh[...]
    )
    o[...] = (1.0 - z) * ht


_tc_combine = pl.pallas_call(
    _tc_body,
    grid=(NP // BN,),
    in_specs=[
        pl.BlockSpec((NC, BN, D), lambda i: (0, i, 0)),
        pl.BlockSpec((NC, BN, D), lambda i: (0, i, 0)),
        pl.BlockSpec((BN, D), lambda i: (i, 0)),
        pl.BlockSpec((D, D), lambda i: (0, 0)),
        pl.BlockSpec((D, D), lambda i: (0, 0)),
        pl.BlockSpec((1, D), lambda i: (0, 0)),
        pl.BlockSpec((D, D), lambda i: (0, 0)),
        pl.BlockSpec((D, D), lambda i: (0, 0)),
        pl.BlockSpec((1, D), lambda i: (0, 0)),
    ],
    out_specs=pl.BlockSpec((BN, D), lambda i: (i, 0)),
    out_shape=jax.ShapeDtypeStruct((NP, D), jnp.float32),
)


def kernel(X, edge_index, W_rel_z, W_root_z, b_z, W_rel_r, W_root_r, b_r,
           W_rel_h, W_root_h, b_h):
    src = edge_index[0].reshape(NW * NSB, SBC, CE)
    dst = edge_index[1].reshape(NW * NSB, SBC, CE)
    zsum = jnp.zeros((ZR, D), jnp.float32)
    ones = jnp.ones((CE, D), jnp.float32)
    x_pad = jnp.concatenate([X, jnp.zeros((NP - N, D), jnp.float32)])
    psum, pcnt = _sc_aggregate(x_pad, src, dst, zsum, ones)
    out = _tc_combine(
        psum, pcnt, x_pad,
        W_rel_z[:D], W_root_z[:D], b_z.reshape(1, D),
        W_rel_h[:D], W_root_h[:D], b_h.reshape(1, D),
    )
    return out[:N]
